# Initial kernel scaffold; baseline (speedup 1.0000x reference)
import jax
import jax.numpy as jnp
from jax.experimental import pallas as pl


def kernel(x, edge_index, W1, b1, W2, b2):
    raise NotImplementedError("write your pallas kernel here")


# speedup vs baseline: 8.7840x; 8.7840x over previous
import functools

import jax
import jax.numpy as jnp
from jax import lax
from jax.experimental import pallas as pl
from jax.experimental.pallas import tpu as pltpu
from jax.experimental.pallas import tpu_sc as plsc

N_PAD = 10240
D = 128
NC = 2
NS = 16
NW = NC * NS
E_PAD = 327680
CHUNKS = 80
CL = 128
RPT = N_PAD // NS
BLK = 1024
GRID = N_PAD // BLK


def _sc_mesh():
    return plsc.VectorSubcoreMesh(core_axis_name="c", subcore_axis_name="s",
                                  num_cores=NC, num_subcores=NS)


def _deg_body(dst_hbm, zeros1_hbm, out_hbm, acc, idx_v, ones_v):
    c = lax.axis_index("c")
    s = lax.axis_index("s")
    wid = c * NS + s
    pltpu.sync_copy(dst_hbm.at[wid], idx_v)
    for i in range(CL // 16):
        ones_v[pl.ds(i * 16, 16)] = jnp.ones((16,), jnp.float32)
    pltpu.sync_copy(zeros1_hbm.at[pl.ds(s * RPT, RPT)], acc.at[pl.ds(s * RPT, RPT)])
    plsc.subcore_barrier()

    def body(j, carry):
        pltpu.sync_copy(ones_v, acc.at[idx_v.at[j]], add=True)
        return carry

    lax.fori_loop(0, CHUNKS, body, 0)
    plsc.subcore_barrier()
    pltpu.sync_copy(acc.at[pl.ds(s * RPT, RPT)], out_hbm.at[c, pl.ds(s * RPT, RPT)])


def _deg_call(dst, zeros1):
    f = pl.kernel(
        _deg_body,
        out_type=jax.ShapeDtypeStruct((NC, N_PAD), jnp.float32),
        mesh=_sc_mesh(),
        scratch_types=[
            pltpu.VMEM_SHARED((N_PAD,), jnp.float32),
            pltpu.VMEM((CHUNKS, CL), jnp.int32),
            pltpu.VMEM((CL,), jnp.float32),
        ],
    )
    return f(dst, zeros1)


def _spmm_body(y_hbm, src_hbm, dst_hbm, zeros_hbm, out_hbm,
               acc, si_v, di_v, rows_v, sem):
    c = lax.axis_index("c")
    s = lax.axis_index("s")
    wid = c * NS + s
    pltpu.sync_copy(src_hbm.at[wid], si_v)
    pltpu.sync_copy(dst_hbm.at[wid], di_v)
    pltpu.sync_copy(zeros_hbm.at[pl.ds(s * RPT, RPT)], acc.at[pl.ds(s * RPT, RPT)])
    plsc.subcore_barrier()

    def body(j, carry):
        pltpu.async_copy(y_hbm.at[si_v.at[j]], rows_v, sem).wait()
        pltpu.sync_copy(rows_v, acc.at[di_v.at[j]], add=True)
        return carry

    lax.fori_loop(0, CHUNKS, body, 0)
    plsc.subcore_barrier()
    pltpu.sync_copy(acc.at[pl.ds(s * RPT, RPT)], out_hbm.at[c, pl.ds(s * RPT, RPT)])


def _spmm_call(y, src, dst, zeros_nd):
    f = pl.kernel(
        _spmm_body,
        out_type=jax.ShapeDtypeStruct((NC, N_PAD, D), jnp.float32),
        mesh=_sc_mesh(),
        scratch_types=[
            pltpu.VMEM_SHARED((N_PAD, D), jnp.float32),
            pltpu.VMEM((CHUNKS, CL), jnp.int32),
            pltpu.VMEM((CHUNKS, CL), jnp.int32),
            pltpu.VMEM((CL, D), jnp.float32),
            pltpu.SemaphoreType.DMA,
        ],
    )
    return f(y, src, dst, zeros_nd)


def _prep_body(deg_ref, x_ref, dinv_ref, y1_ref):
    d = deg_ref[0] + deg_ref[1] + 1.0
    dinv = lax.rsqrt(d)
    dinv_ref[...] = dinv
    y1_ref[...] = x_ref[...] * dinv


def _prep_call(deg2, xp):
    return pl.pallas_call(
        _prep_body,
        grid=(GRID,),
        in_specs=[
            pl.BlockSpec((NC, BLK, 1), lambda i: (0, i, 0)),
            pl.BlockSpec((BLK, D), lambda i: (i, 0)),
        ],
        out_specs=[
            pl.BlockSpec((BLK, 1), lambda i: (i, 0)),
            pl.BlockSpec((BLK, D), lambda i: (i, 0)),
        ],
        out_shape=[
            jax.ShapeDtypeStruct((N_PAD, 1), jnp.float32),
            jax.ShapeDtypeStruct((N_PAD, D), jnp.float32),
        ],
    )(deg2, xp)


def _mid_body(z_ref, y1_ref, dinv_ref, w1_ref, b1_ref, w2_ref, y2_ref):
    s1 = (z_ref[0] + z_ref[1] + y1_ref[...]) * dinv_ref[...]
    h = jnp.dot(s1, w1_ref[...], preferred_element_type=jnp.float32) + b1_ref[...]
    h = jnp.maximum(h, 0.0)
    y2_ref[...] = jnp.dot(h, w2_ref[...],
                          preferred_element_type=jnp.float32) * dinv_ref[...]


def _mid_call(z1, y1, dinv, W1, b1, W2):
    c2 = W1.shape[1]
    return pl.pallas_call(
        _mid_body,
        grid=(GRID,),
        in_specs=[
            pl.BlockSpec((NC, BLK, D), lambda i: (0, i, 0)),
            pl.BlockSpec((BLK, D), lambda i: (i, 0)),
            pl.BlockSpec((BLK, 1), lambda i: (i, 0)),
            pl.BlockSpec((D, c2), lambda i: (0, 0)),
            pl.BlockSpec((1, c2), lambda i: (0, 0)),
            pl.BlockSpec((c2, D), lambda i: (0, 0)),
        ],
        out_specs=pl.BlockSpec((BLK, D), lambda i: (i, 0)),
        out_shape=jax.ShapeDtypeStruct((N_PAD, D), jnp.float32),
    )(z1, y1, dinv, W1, b1, W2)


def _fin_body(z_ref, y2_ref, dinv_ref, b2_ref, o_ref):
    o_ref[...] = (z_ref[0] + z_ref[1] + y2_ref[...]) * dinv_ref[...] + b2_ref[...]


def _fin_call(z2, y2, dinv, b2):
    return pl.pallas_call(
        _fin_body,
        grid=(GRID,),
        in_specs=[
            pl.BlockSpec((NC, BLK, D), lambda i: (0, i, 0)),
            pl.BlockSpec((BLK, D), lambda i: (i, 0)),
            pl.BlockSpec((BLK, 1), lambda i: (i, 0)),
            pl.BlockSpec((1, D), lambda i: (0, 0)),
        ],
        out_specs=pl.BlockSpec((BLK, D), lambda i: (i, 0)),
        out_shape=jax.ShapeDtypeStruct((N_PAD, D), jnp.float32),
    )(z2, y2, dinv, b2)


def kernel(x, edge_index, W1, b1, W2, b2):
    n = x.shape[0]
    e = edge_index.shape[1]
    src = edge_index[0].astype(jnp.int32)
    dst = edge_index[1].astype(jnp.int32)
    src = jnp.pad(src, (0, E_PAD - e), constant_values=N_PAD - 1)
    dst = jnp.pad(dst, (0, E_PAD - e), constant_values=N_PAD - 1)
    src = src.reshape(NW, CHUNKS, CL)
    dst = dst.reshape(NW, CHUNKS, CL)
    xp = jnp.pad(x, ((0, N_PAD - n), (0, 0)))
    zeros1 = jnp.zeros((N_PAD,), jnp.float32)
    zeros_nd = jnp.zeros((N_PAD, D), jnp.float32)

    deg2 = _deg_call(dst, zeros1)
    dinv, y1 = _prep_call(deg2.reshape(NC, N_PAD, 1), xp)
    z1 = _spmm_call(y1, src, dst, zeros_nd)
    y2 = _mid_call(z1, y1, dinv, W1, b1.reshape(1, -1), W2)
    z2 = _spmm_call(y2, src, dst, zeros_nd)
    out = _fin_call(z2, y2, dinv, b2.reshape(1, -1))
    return out[:n]


# speedup vs baseline: 9.3849x; 1.0684x over previous
import functools

import jax
import jax.numpy as jnp
from jax import lax
from jax.experimental import pallas as pl
from jax.experimental.pallas import tpu as pltpu
from jax.experimental.pallas import tpu_sc as plsc

N_PAD = 10240
D = 128
NC = 2
NS = 16
NW = NC * NS
E_PAD = 327680
CHUNKS = 80
HALF = 40
CL = 128
RPT = N_PAD // NS
BLK = 1024
GRID = N_PAD // BLK


def _sc_mesh():
    return plsc.VectorSubcoreMesh(core_axis_name="c", subcore_axis_name="s",
                                  num_cores=NC, num_subcores=NS)


def _deg_body(dst_hbm, zeros1_hbm, out_hbm, acc, idx_v, ones_v):
    c = lax.axis_index("c")
    s = lax.axis_index("s")
    wid = c * NS + s
    pltpu.sync_copy(dst_hbm.at[wid], idx_v)
    for i in range(CL // 16):
        ones_v[pl.ds(i * 16, 16)] = jnp.ones((16,), jnp.float32)
    pltpu.sync_copy(zeros1_hbm.at[pl.ds(s * RPT, RPT)], acc.at[pl.ds(s * RPT, RPT)])
    plsc.subcore_barrier()

    def body(j, carry):
        pltpu.sync_copy(ones_v, acc.at[idx_v.at[j]], add=True)
        return carry

    lax.fori_loop(0, CHUNKS, body, 0)
    plsc.subcore_barrier()
    pltpu.sync_copy(acc.at[pl.ds(s * RPT, RPT)], out_hbm.at[c, pl.ds(s * RPT, RPT)])


def _deg_call(dst, zeros1):
    f = pl.kernel(
        _deg_body,
        out_type=jax.ShapeDtypeStruct((NC, N_PAD), jnp.float32),
        mesh=_sc_mesh(),
        scratch_types=[
            pltpu.VMEM_SHARED((N_PAD,), jnp.float32),
            pltpu.VMEM((CHUNKS, CL), jnp.int32),
            pltpu.VMEM((CL,), jnp.float32),
        ],
    )
    return f(dst, zeros1)


NBUF = 2


def _spmm_body(y_hbm, src_hbm, dst_hbm, zeros_hbm, out_hbm,
               acc, si_v, di_v, r0, r1, gsem, ssem):
    rows = [r0, r1]
    c = lax.axis_index("c")
    s = lax.axis_index("s")
    wid = c * NS + s
    pltpu.sync_copy(zeros_hbm.at[pl.ds(s * RPT, RPT)], acc.at[pl.ds(s * RPT, RPT)])
    plsc.subcore_barrier()

    for h in range(CHUNKS // HALF):
        pltpu.sync_copy(src_hbm.at[wid, pl.ds(h * HALF, HALF)], si_v)
        pltpu.sync_copy(dst_hbm.at[wid, pl.ds(h * HALF, HALF)], di_v)
        for b in range(NBUF):
            pltpu.async_copy(y_hbm.at[si_v.at[b]], rows[b], gsem.at[b])

        def body(i, carry):
            for b in range(NBUF):
                j = i * NBUF + b
                pltpu.make_async_copy(y_hbm.at[si_v.at[j]], rows[b],
                                      gsem.at[b]).wait()
                pltpu.async_copy(rows[b], acc.at[di_v.at[j]], ssem.at[b],
                                 add=True)
                bp = (b - 1) % NBUF
                jp = j - 1
                tgt = jp + NBUF

                @pl.when((jp >= 0) & (tgt < HALF))
                def _():
                    pltpu.make_async_copy(rows[bp], acc.at[di_v.at[jp]],
                                          ssem.at[bp]).wait()
                    pltpu.async_copy(y_hbm.at[si_v.at[tgt]], rows[bp],
                                     gsem.at[bp])
            return carry

        lax.fori_loop(0, HALF // NBUF, body, 0)
        for b in range(NBUF):
            jp = HALF - NBUF + b
            pltpu.make_async_copy(rows[b], acc.at[di_v.at[jp]],
                                  ssem.at[b]).wait()
    plsc.subcore_barrier()
    pltpu.sync_copy(acc.at[pl.ds(s * RPT, RPT)], out_hbm.at[c, pl.ds(s * RPT, RPT)])


def _spmm_call(y, src, dst, zeros_nd):
    f = pl.kernel(
        _spmm_body,
        out_type=jax.ShapeDtypeStruct((NC, N_PAD, D), jnp.float32),
        mesh=_sc_mesh(),
        scratch_types=[
            pltpu.VMEM_SHARED((N_PAD, D), jnp.float32),
            pltpu.VMEM((HALF, CL), jnp.int32),
            pltpu.VMEM((HALF, CL), jnp.int32),
            pltpu.VMEM((CL, D), jnp.float32),
            pltpu.VMEM((CL, D), jnp.float32),
            pltpu.SemaphoreType.DMA((NBUF,)),
            pltpu.SemaphoreType.DMA((NBUF,)),
        ],
    )
    return f(y, src, dst, zeros_nd)


def _prep_body(deg_ref, x_ref, dinv_ref, y1_ref):
    d = deg_ref[0] + deg_ref[1] + 1.0
    dinv = lax.rsqrt(d)
    dinv_ref[...] = dinv
    y1_ref[...] = x_ref[...] * dinv


def _prep_call(deg2, xp):
    return pl.pallas_call(
        _prep_body,
        grid=(GRID,),
        in_specs=[
            pl.BlockSpec((NC, BLK, 1), lambda i: (0, i, 0)),
            pl.BlockSpec((BLK, D), lambda i: (i, 0)),
        ],
        out_specs=[
            pl.BlockSpec((BLK, 1), lambda i: (i, 0)),
            pl.BlockSpec((BLK, D), lambda i: (i, 0)),
        ],
        out_shape=[
            jax.ShapeDtypeStruct((N_PAD, 1), jnp.float32),
            jax.ShapeDtypeStruct((N_PAD, D), jnp.float32),
        ],
    )(deg2, xp)


def _mid_body(z_ref, y1_ref, dinv_ref, w1_ref, b1_ref, w2_ref, y2_ref):
    s1 = (z_ref[0] + z_ref[1] + y1_ref[...]) * dinv_ref[...]
    h = jnp.dot(s1, w1_ref[...], preferred_element_type=jnp.float32) + b1_ref[...]
    h = jnp.maximum(h, 0.0)
    y2_ref[...] = jnp.dot(h, w2_ref[...],
                          preferred_element_type=jnp.float32) * dinv_ref[...]


def _mid_call(z1, y1, dinv, W1, b1, W2):
    c2 = W1.shape[1]
    return pl.pallas_call(
        _mid_body,
        grid=(GRID,),
        in_specs=[
            pl.BlockSpec((NC, BLK, D), lambda i: (0, i, 0)),
            pl.BlockSpec((BLK, D), lambda i: (i, 0)),
            pl.BlockSpec((BLK, 1), lambda i: (i, 0)),
            pl.BlockSpec((D, c2), lambda i: (0, 0)),
            pl.BlockSpec((1, c2), lambda i: (0, 0)),
            pl.BlockSpec((c2, D), lambda i: (0, 0)),
        ],
        out_specs=pl.BlockSpec((BLK, D), lambda i: (i, 0)),
        out_shape=jax.ShapeDtypeStruct((N_PAD, D), jnp.float32),
    )(z1, y1, dinv, W1, b1, W2)


def _fin_body(z_ref, y2_ref, dinv_ref, b2_ref, o_ref):
    o_ref[...] = (z_ref[0] + z_ref[1] + y2_ref[...]) * dinv_ref[...] + b2_ref[...]


def _fin_call(z2, y2, dinv, b2):
    return pl.pallas_call(
        _fin_body,
        grid=(GRID,),
        in_specs=[
            pl.BlockSpec((NC, BLK, D), lambda i: (0, i, 0)),
            pl.BlockSpec((BLK, D), lambda i: (i, 0)),
            pl.BlockSpec((BLK, 1), lambda i: (i, 0)),
            pl.BlockSpec((1, D), lambda i: (0, 0)),
        ],
        out_specs=pl.BlockSpec((BLK, D), lambda i: (i, 0)),
        out_shape=jax.ShapeDtypeStruct((N_PAD, D), jnp.float32),
    )(z2, y2, dinv, b2)


def kernel(x, edge_index, W1, b1, W2, b2):
    n = x.shape[0]
    e = edge_index.shape[1]
    src = edge_index[0].astype(jnp.int32)
    dst = edge_index[1].astype(jnp.int32)
    src = jnp.pad(src, (0, E_PAD - e), constant_values=N_PAD - 1)
    dst = jnp.pad(dst, (0, E_PAD - e), constant_values=N_PAD - 1)
    src = src.reshape(NW, CHUNKS, CL)
    dst = dst.reshape(NW, CHUNKS, CL)
    xp = jnp.pad(x, ((0, N_PAD - n), (0, 0)))
    zeros1 = jnp.zeros((N_PAD,), jnp.float32)
    zeros_nd = jnp.zeros((N_PAD, D), jnp.float32)

    deg2 = _deg_call(dst, zeros1)
    dinv, y1 = _prep_call(deg2.reshape(NC, N_PAD, 1), xp)
    z1 = _spmm_call(y1, src, dst, zeros_nd)
    y2 = _mid_call(z1, y1, dinv, W1, b1.reshape(1, -1), W2)
    z2 = _spmm_call(y2, src, dst, zeros_nd)
    out = _fin_call(z2, y2, dinv, b2.reshape(1, -1))
    return out[:n]


# speedup vs baseline: 13.0530x; 1.3908x over previous
import functools

import jax
import jax.numpy as jnp
from jax import lax
from jax.experimental import pallas as pl
from jax.experimental.pallas import tpu as pltpu
from jax.experimental.pallas import tpu_sc as plsc

N_PAD = 10240
D = 128
NC = 2
NS = 16
NW = NC * NS
CL = 128
CH_TOT = 2560
E_PAD = CH_TOT * CL
CH_EXTRA = 40
HALF = 40
CNT0 = 144
CNT1 = 16
MAXST = 4
RPT = N_PAD // NS
BLK = 1024
GRID = N_PAD // BLK


def _sc_mesh():
    return plsc.VectorSubcoreMesh(core_axis_name="c", subcore_axis_name="s",
                                  num_cores=NC, num_subcores=NS)


CH_W = CH_TOT // NW


def _deg_body(dst_hbm, zeros1_hbm, out_hbm, acc, idx_v, ones_v):
    c = lax.axis_index("c")
    s = lax.axis_index("s")
    wid = c * NS + s
    pltpu.sync_copy(dst_hbm.at[pl.ds(wid * CH_W, CH_W)], idx_v)
    for i in range(CL // 16):
        ones_v[pl.ds(i * 16, 16)] = jnp.ones((16,), jnp.float32)
    pltpu.sync_copy(zeros1_hbm.at[pl.ds(s * RPT, RPT)], acc.at[pl.ds(s * RPT, RPT)])
    plsc.subcore_barrier()

    def body(j, carry):
        pltpu.sync_copy(ones_v, acc.at[idx_v.at[j]], add=True)
        return carry

    lax.fori_loop(0, CH_W, body, 0)
    plsc.subcore_barrier()
    pltpu.sync_copy(acc.at[pl.ds(s * RPT, RPT)], out_hbm.at[c, pl.ds(s * RPT, RPT)])


def _deg_call(dst, zeros1):
    f = pl.kernel(
        _deg_body,
        out_type=jax.ShapeDtypeStruct((NC, N_PAD), jnp.float32),
        mesh=_sc_mesh(),
        scratch_types=[
            pltpu.VMEM_SHARED((N_PAD,), jnp.float32),
            pltpu.VMEM((CH_W, CL), jnp.int32),
            pltpu.VMEM((CL,), jnp.float32),
        ],
    )
    return f(dst, zeros1)


NBUF = 2


def _spmm_body(y_hbm, src_hbm, dst_hbm, out_hbm,
               acc, si_v, di_v, r0, r1, gsem, ssem):
    rows = [r0, r1]
    c = lax.axis_index("c")
    s = lax.axis_index("s")
    cnt = jnp.where(c == 0, CNT0, CNT1)
    base_w = c * (NS * CNT0) + s * cnt

    def zbody(i, carry):
        for k in range(D // 16):
            r0[i, pl.ds(k * 16, 16)] = jnp.zeros((16,), jnp.float32)
        return carry

    lax.fori_loop(0, CL, zbody, 0)
    for k in range(RPT // CL):
        pltpu.sync_copy(r0, acc.at[pl.ds(s * RPT + k * CL, CL)])
    plsc.subcore_barrier()

    for st in range(MAXST):
        done = st * HALF

        @pl.when(done < cnt)
        def _stage():
            pcount = jnp.minimum(HALF, cnt - done)
            off = pl.multiple_of(base_w + done, 8)
            pltpu.sync_copy(src_hbm.at[pl.ds(off, HALF)], si_v)
            pltpu.sync_copy(dst_hbm.at[pl.ds(off, HALF)], di_v)
            for b in range(NBUF):
                pltpu.async_copy(y_hbm.at[si_v.at[b]], rows[b], gsem.at[b])

            def body(i, carry):
                for b in range(NBUF):
                    j = i * NBUF + b
                    pltpu.make_async_copy(y_hbm.at[si_v.at[j]], rows[b],
                                          gsem.at[b]).wait()
                    pltpu.async_copy(rows[b], acc.at[di_v.at[j]], ssem.at[b],
                                     add=True)
                    bp = (b - 1) % NBUF
                    jp = j - 1
                    tgt = jp + NBUF

                    @pl.when((jp >= 0) & (tgt < pcount))
                    def _():
                        pltpu.make_async_copy(rows[bp], acc.at[di_v.at[jp]],
                                              ssem.at[bp]).wait()
                        pltpu.async_copy(y_hbm.at[si_v.at[tgt]], rows[bp],
                                         gsem.at[bp])
                return carry

            lax.fori_loop(0, pcount // NBUF, body, 0)
            for b in range(NBUF):
                jp = pcount - NBUF + b
                pltpu.make_async_copy(rows[b], acc.at[di_v.at[jp]],
                                      ssem.at[b]).wait()
    plsc.subcore_barrier()
    pltpu.sync_copy(acc.at[pl.ds(s * RPT, RPT)], out_hbm.at[c, pl.ds(s * RPT, RPT)])


def _spmm_call(y, src, dst):
    f = pl.kernel(
        _spmm_body,
        out_type=jax.ShapeDtypeStruct((NC, N_PAD, D), jnp.float32),
        mesh=_sc_mesh(),
        scratch_types=[
            pltpu.VMEM_SHARED((N_PAD, D), jnp.float32),
            pltpu.VMEM((HALF, CL), jnp.int32),
            pltpu.VMEM((HALF, CL), jnp.int32),
            pltpu.VMEM((CL, D), jnp.float32),
            pltpu.VMEM((CL, D), jnp.float32),
            pltpu.SemaphoreType.DMA((NBUF,)),
            pltpu.SemaphoreType.DMA((NBUF,)),
        ],
    )
    return f(y, src, dst)


def _prep_body(deg_ref, x_ref, dinv_ref, y1_ref):
    d = deg_ref[0] + deg_ref[1] + 1.0
    dinv = lax.rsqrt(d)
    dinv_ref[...] = dinv
    y1_ref[...] = x_ref[...] * dinv


def _prep_call(deg2, xp):
    return pl.pallas_call(
        _prep_body,
        grid=(GRID,),
        in_specs=[
            pl.BlockSpec((NC, BLK, 1), lambda i: (0, i, 0)),
            pl.BlockSpec((BLK, D), lambda i: (i, 0)),
        ],
        out_specs=[
            pl.BlockSpec((BLK, 1), lambda i: (i, 0)),
            pl.BlockSpec((BLK, D), lambda i: (i, 0)),
        ],
        out_shape=[
            jax.ShapeDtypeStruct((N_PAD, 1), jnp.float32),
            jax.ShapeDtypeStruct((N_PAD, D), jnp.float32),
        ],
    )(deg2, xp)


def _mid_body(z_ref, y1_ref, dinv_ref, w1_ref, b1_ref, w2_ref, y2_ref):
    s1 = (z_ref[0] + z_ref[1] + y1_ref[...]) * dinv_ref[...]
    h = jnp.dot(s1, w1_ref[...], preferred_element_type=jnp.float32) + b1_ref[...]
    h = jnp.maximum(h, 0.0)
    y2_ref[...] = jnp.dot(h, w2_ref[...],
                          preferred_element_type=jnp.float32) * dinv_ref[...]


def _mid_call(z1, y1, dinv, W1, b1, W2):
    c2 = W1.shape[1]
    return pl.pallas_call(
        _mid_body,
        grid=(GRID,),
        in_specs=[
            pl.BlockSpec((NC, BLK, D), lambda i: (0, i, 0)),
            pl.BlockSpec((BLK, D), lambda i: (i, 0)),
            pl.BlockSpec((BLK, 1), lambda i: (i, 0)),
            pl.BlockSpec((D, c2), lambda i: (0, 0)),
            pl.BlockSpec((1, c2), lambda i: (0, 0)),
            pl.BlockSpec((c2, D), lambda i: (0, 0)),
        ],
        out_specs=pl.BlockSpec((BLK, D), lambda i: (i, 0)),
        out_shape=jax.ShapeDtypeStruct((N_PAD, D), jnp.float32),
    )(z1, y1, dinv, W1, b1, W2)


def _fin_body(z_ref, y2_ref, dinv_ref, b2_ref, o_ref):
    o_ref[...] = (z_ref[0] + z_ref[1] + y2_ref[...]) * dinv_ref[...] + b2_ref[...]


def _fin_call(z2, y2, dinv, b2, n):
    fblk = n // GRID
    return pl.pallas_call(
        _fin_body,
        grid=(GRID,),
        in_specs=[
            pl.BlockSpec((NC, fblk, D), lambda i: (0, i, 0)),
            pl.BlockSpec((fblk, D), lambda i: (i, 0)),
            pl.BlockSpec((fblk, 1), lambda i: (i, 0)),
            pl.BlockSpec((1, D), lambda i: (0, 0)),
        ],
        out_specs=pl.BlockSpec((fblk, D), lambda i: (i, 0)),
        out_shape=jax.ShapeDtypeStruct((n, D), jnp.float32),
    )(z2, y2, dinv, b2)


def kernel(x, edge_index, W1, b1, W2, b2):
    n = x.shape[0]
    e = edge_index.shape[1]
    src = edge_index[0].astype(jnp.int32)
    dst = edge_index[1].astype(jnp.int32)
    tot = (CH_TOT + CH_EXTRA) * CL
    src = jnp.pad(src, (0, tot - e), constant_values=N_PAD - 1).reshape(-1, CL)
    dst = jnp.pad(dst, (0, tot - e), constant_values=N_PAD - 1).reshape(-1, CL)
    xp = jnp.pad(x, ((0, N_PAD - n), (0, 0)))
    zeros1 = jnp.zeros((N_PAD,), jnp.float32)

    deg2 = _deg_call(dst, zeros1)
    dinv, y1 = _prep_call(deg2.reshape(NC, N_PAD, 1), xp)
    z1 = _spmm_call(y1, src, dst)
    y2 = _mid_call(z1, y1, dinv, W1, b1.reshape(1, -1), W2)
    z2 = _spmm_call(y2, src, dst)
    return _fin_call(z2, y2, dinv, b2.reshape(1, -1), n)
